# Initial kernel scaffold; baseline (speedup 1.0000x reference)
import jax
import jax.numpy as jnp
from jax.experimental import pallas as pl


def kernel(x, W1, b1, W2, b2):
    raise NotImplementedError("write your pallas kernel here")


# speedup vs baseline: 1.8182x; 1.8182x over previous
import functools

import jax
import jax.numpy as jnp
from jax import lax
from jax.experimental import pallas as pl
from jax.experimental.pallas import tpu as pltpu
from jax.experimental.pallas import tpu_sc as plsc

B, N, D, H = 16, 4096, 512, 1024
K = N // 2
TM = 1024

NC, NS = 2, 16
NW = NC * NS
ROWS_PER_W = (B * K) // NW
CH = 64


def _score_body(x_ref, w1_ref, b1_ref, w2_ref, b2_ref, att_ref):
    z = jnp.dot(x_ref[...], w1_ref[...], preferred_element_type=jnp.float32)
    z = z + b1_ref[...]
    h = jnp.where(z >= 0, z, 0.2 * z)
    att = lax.dot_general(w2_ref[...], h, (((1,), (1,)), ((), ())),
                          preferred_element_type=jnp.float32)
    att_ref[...] = att + b2_ref[...]


def _score(x2, W1, b1, W2, b2):
    return pl.pallas_call(
        _score_body,
        grid=(B * N // TM,),
        in_specs=[
            pl.BlockSpec((TM, D), lambda i: (i, 0)),
            pl.BlockSpec((D, H), lambda i: (0, 0)),
            pl.BlockSpec((1, H), lambda i: (0, 0)),
            pl.BlockSpec((1, H), lambda i: (0, 0)),
            pl.BlockSpec((1, 1), lambda i: (0, 0)),
        ],
        out_specs=pl.BlockSpec((1, TM), lambda i: (0, i)),
        out_shape=jax.ShapeDtypeStruct((1, B * N), jnp.float32),
    )(x2, W1, b1.reshape(1, H), W2.reshape(1, H), b2.reshape(1, 1))


def _roll(v, j):
    return jnp.concatenate([v[:, j:], v[:, :j]], axis=1)


def _sort_body(att_ref, idx_ref):
    v = att_ref[...]
    ic = lax.broadcasted_iota(jnp.int32, (B, N), 1)
    ii = ic
    k = 2
    while k <= N:
        j = k // 2
        while j >= 1:
            islo = (ii & j) == 0
            vp = jnp.where(islo, _roll(v, j), _roll(v, N - j))
            ip = jnp.where(islo, _roll(ic, j), _roll(ic, N - j))
            pred = (v > vp) | ((v == vp) & (ic < ip))
            d = (ii & k) == 0
            take_self = (islo == d) == pred
            v = jnp.where(take_self, v, vp)
            ic = jnp.where(take_self, ic, ip)
            j //= 2
        k *= 2
    base = lax.broadcasted_iota(jnp.int32, (B, K), 0) * N
    idx_ref[...] = ic[:, :K] + base


def _sort(att):
    return pl.pallas_call(
        _sort_body,
        out_shape=jax.ShapeDtypeStruct((B, K), jnp.int32),
    )(att)


def _gather_body(x_hbm, idx_hbm, out_hbm, idx_v, rows_v, sem):
    wid = lax.axis_index("s") * NC + lax.axis_index("c")
    base = wid * ROWS_PER_W

    def chunk(i, carry):
        off = base + i * CH
        pltpu.sync_copy(idx_hbm.at[pl.ds(off, CH)], idx_v)
        pltpu.async_copy(x_hbm.at[idx_v], rows_v, sem).wait()
        pltpu.sync_copy(rows_v, out_hbm.at[pl.ds(off, CH)])
        return carry

    lax.fori_loop(0, ROWS_PER_W // CH, chunk, 0)


def _gather(x2, gidx):
    f = pl.kernel(
        _gather_body,
        mesh=plsc.VectorSubcoreMesh(core_axis_name="c", subcore_axis_name="s"),
        out_type=jax.ShapeDtypeStruct((B * K, D), jnp.float32),
        scratch_types=[
            pltpu.VMEM((CH,), jnp.int32),
            pltpu.VMEM((CH, D), jnp.float32),
            pltpu.SemaphoreType.DMA,
        ],
    )
    return f(x2, gidx)


def kernel(x, W1, b1, W2, b2):
    x2 = x.reshape(B * N, D)
    att = _score(x2, W1, b1, W2, b2).reshape(B, N)
    gidx = _sort(att)
    sel = _gather(x2, gidx.reshape(B * K))
    return sel.reshape(B, K, D), att
